# Initial kernel scaffold; baseline (speedup 1.0000x reference)
import jax
import jax.numpy as jnp
from jax.experimental import pallas as pl


def kernel(x, edge_index, edge_attr, eps, W1, b1, bn1_g, bn1_b, W2, b2, bond_emb, bn_g, bn_b):
    raise NotImplementedError("write your pallas kernel here")


# speedup vs baseline: 6.0807x; 6.0807x over previous
import functools

import jax
import jax.numpy as jnp
from jax import lax
from jax.experimental import pallas as pl
from jax.experimental.pallas import tpu as pltpu
from jax.experimental.pallas import tpu_sc as plsc

N = 10000
E = 320000
D = 128
L = 3
NF = 3
V = 8

NC = 2
NS = 16
NW = NC * NS
EPW = E // NW
K = 128
NCH = -(-EPW // K)
EPWP = NCH * K
AGG_ROWS = 10240
ZPR = AGG_ROWS // NS
CPR = 1000

_mesh = plsc.VectorSubcoreMesh(core_axis_name="c", subcore_axis_name="s",
                               num_cores=NC, num_subcores=NS)


@functools.partial(
    pl.kernel,
    out_type=jax.ShapeDtypeStruct((NC, N, D), jnp.float32),
    mesh=_mesh,
    scratch_types=[
        pltpu.VMEM((3, K), jnp.int32),
        pltpu.VMEM((K, D), jnp.float32),
        pltpu.VMEM((K, D), jnp.float32),
        pltpu.VMEM_SHARED((AGG_ROWS, D), jnp.float32),
        pltpu.SemaphoreType.DMA,
        pltpu.SemaphoreType.DMA,
    ],
)
def _sc_message_pass(idx3_hbm, h_hbm, ctab_hbm, zeros_hbm, out_hbm,
                     idx_v, hrow_v, crow_v, agg_sh, sem1, sem2):
    c = lax.axis_index("c")
    s = lax.axis_index("s")
    wid = c * NS + s

    r0 = s * ZPR
    pltpu.sync_copy(zeros_hbm.at[pl.ds(r0, ZPR)], agg_sh.at[pl.ds(r0, ZPR)])

    plsc.subcore_barrier()

    def chunk(i, carry):
        pltpu.sync_copy(idx3_hbm.at[wid, :, pl.ds(i * K, K)], idx_v)
        cp_h = pltpu.async_copy(h_hbm.at[idx_v.at[0]], hrow_v, sem1)
        cp_c = pltpu.async_copy(ctab_hbm.at[idx_v.at[1]], crow_v, sem2)
        cp_h.wait()
        cp_c.wait()

        def row(e, carry2):
            for j in range(D // 16):
                sl = pl.ds(j * 16, 16)
                hrow_v[e, sl] = jnp.maximum(hrow_v[e, sl] + crow_v[e, sl], 0.0)
            return carry2

        lax.fori_loop(0, K, row, 0)
        pltpu.sync_copy(hrow_v, agg_sh.at[idx_v.at[2]], add=True)
        return carry

    lax.fori_loop(0, NCH, chunk, 0)
    plsc.subcore_barrier()

    @pl.when(s < N // CPR)
    def _out():
        r0 = s * CPR
        pltpu.sync_copy(agg_sh.at[pl.ds(r0, CPR)], out_hbm.at[c, pl.ds(r0, CPR)])


def _mlp_body(h_ref, agg_ref, eps_ref, w1_ref, b1_ref, g1_ref, bb1_ref,
              w2_ref, b2_ref, g2_ref, bb2_ref, out_ref, *, final_relu):
    h = h_ref[...]
    pre = (1.0 + eps_ref[0, 0]) * h + agg_ref[0] + agg_ref[1]
    t = jnp.dot(pre, w1_ref[...], preferred_element_type=jnp.float32) + b1_ref[...]
    m = jnp.mean(t, axis=0, keepdims=True)
    v = jnp.mean((t - m) ** 2, axis=0, keepdims=True)
    t = (t - m) * lax.rsqrt(v + 1e-5) * g1_ref[...] + bb1_ref[...]
    t = jnp.maximum(t, 0.0)
    t2 = jnp.dot(t, w2_ref[...], preferred_element_type=jnp.float32) + b2_ref[...]
    m2 = jnp.mean(t2, axis=0, keepdims=True)
    v2 = jnp.mean((t2 - m2) ** 2, axis=0, keepdims=True)
    t2 = (t2 - m2) * lax.rsqrt(v2 + 1e-5) * g2_ref[...] + bb2_ref[...]
    if final_relu:
        t2 = jnp.maximum(t2, 0.0)
    out_ref[...] = t2


def _mlp(h, agg, eps_l, w1, b1, g1, bb1, w2, b2, g2, bb2, final_relu):
    return pl.pallas_call(
        functools.partial(_mlp_body, final_relu=final_relu),
        out_shape=jax.ShapeDtypeStruct((N, D), jnp.float32),
    )(h, agg, eps_l, w1, b1, g1, bb1, w2, b2, g2, bb2)


def kernel(x, edge_index, edge_attr, eps, W1, b1, bn1_g, bn1_b, W2, b2,
           bond_emb, bn_g, bn_b):
    src = edge_index[0]
    dst = edge_index[1]
    code = edge_attr[:, 0] + V * edge_attr[:, 1] + V * V * edge_attr[:, 2]
    npad = EPWP - EPW
    src_p = jnp.concatenate(
        [src.reshape(NW, EPW), jnp.zeros((NW, npad), jnp.int32)], axis=1)
    code_p = jnp.concatenate(
        [code.reshape(NW, EPW), jnp.zeros((NW, npad), jnp.int32)], axis=1)
    dst_p = jnp.concatenate(
        [dst.reshape(NW, EPW), jnp.full((NW, npad), N, jnp.int32)], axis=1)
    idx3 = jnp.stack([src_p, code_p, dst_p], axis=1).astype(jnp.int32)
    ctab = (bond_emb[:, 2][:, :, None, None, :]
            + bond_emb[:, 1][:, None, :, None, :]
            + bond_emb[:, 0][:, None, None, :, :]).reshape(L, V ** NF, D)
    zeros = jnp.zeros((AGG_ROWS, D), jnp.float32)

    h = x
    for l in range(L):
        agg = _sc_message_pass(idx3, h, ctab[l], zeros)
        h = _mlp(h, agg, eps[l].reshape(1, 1),
                 W1[l], b1[l].reshape(1, 2 * D),
                 bn1_g[l].reshape(1, 2 * D), bn1_b[l].reshape(1, 2 * D),
                 W2[l], b2[l].reshape(1, D),
                 bn_g[l].reshape(1, D), bn_b[l].reshape(1, D),
                 final_relu=(l < L - 1))
    return h
